# baseline (speedup 1.0000x reference)
import functools

import jax
import jax.numpy as jnp
from jax import lax
from jax.experimental import pallas as pl
from jax.experimental.pallas import tpu as pltpu
from jax.experimental.pallas import tpu_sc as plsc

_B = 4096
_I = 200
_H = 64
_LANES = 16
_NSLOT = 8
_NH = 32


def _make_sc_gather(h0, nh):
    info = plsc.get_sparse_core_info()
    NC, NS = info.num_cores, info.num_subcores
    NW = NC * NS
    bpw = _B // NW
    mesh = plsc.VectorSubcoreMesh(core_axis_name="c", subcore_axis_name="s")

    @functools.partial(
        pl.kernel,
        mesh=mesh,
        out_type=jax.ShapeDtypeStruct((nh, _B), jnp.float32),
        scratch_types=[
            pltpu.VMEM((bpw,), jnp.int32),
            pltpu.VMEM((_NSLOT, nh, 128), jnp.float32),
            pltpu.VMEM_SHARED((NS, nh, 128), jnp.float32),
            pltpu.SemaphoreType.DMA((_NSLOT,)),
            pltpu.SemaphoreType.DMA((_NSLOT,)),
        ],
    )
    def gather_kernel(
        idx_hbm, tableT_hbm, out_hbm, idx_v, ring_v, sh_v, tsems, csems
    ):
        cid = lax.axis_index("c")
        sid = lax.axis_index("s")
        wid = sid * NC + cid
        base = wid * bpw
        my_sh = sh_v.at[sid]

        pltpu.sync_copy(idx_hbm.at[pl.ds(base, bpw)], idx_v)

        def fire_tile(u, k):
            start = pl.multiple_of((u >> 7) * 128, 128)
            pltpu.async_copy(
                tableT_hbm.at[pl.ds(h0, nh), pl.ds(start, 128)],
                ring_v.at[k],
                tsems.at[k],
            )

        def wait_tile(k):
            pltpu.make_async_copy(
                tableT_hbm.at[pl.ds(h0, nh), pl.ds(0, 128)],
                ring_v.at[k],
                tsems.at[k],
            ).wait()

        def fire_col(u, b, k):
            pltpu.async_copy(
                ring_v.at[k].at[:, pl.ds(u & 127, 1)],
                my_sh.at[:, pl.ds(b, 1)],
                csems.at[k],
            )

        def wait_col(k):
            pltpu.make_async_copy(
                ring_v.at[k].at[:, pl.ds(0, 1)],
                my_sh.at[:, pl.ds(0, 1)],
                csems.at[k],
            ).wait()

        vs = [idx_v[pl.ds(g * _LANES, _LANES)] for g in range(bpw // _LANES)]

        def u(b):
            return vs[b // _LANES][b % _LANES]

        _LAG = 6
        for b in range(bpw + _LAG):
            if b < bpw:
                if b >= _NSLOT:
                    wait_col(b % _NSLOT)
                fire_tile(u(b), b % _NSLOT)
            if b >= _LAG:
                bb = b - _LAG
                wait_tile(bb % _NSLOT)
                fire_col(u(bb), bb, bb % _NSLOT)
        for k in range(_NSLOT):
            wait_col(k)

        pltpu.sync_copy(my_sh, out_hbm.at[:, pl.ds(base, bpw)])

    return gather_kernel


def _bmm_body0(iembT_ref, uemb_ref, out_ref):
    out_ref[...] = jnp.sum(iembT_ref[...] * uemb_ref[...][None, :, :], axis=1)


def _bmm_body1(iembT_ref, uemb_ref, acc_ref, out_ref):
    out_ref[...] = acc_ref[...] + jnp.sum(
        iembT_ref[...] * uemb_ref[...][None, :, :], axis=1
    )


def _tc_bmm(iembT, uembT, hb, acc=None, block_i=8):
    I, H, B = iembT.shape
    nh = uembT.shape[0]
    in_specs = [
        pl.BlockSpec((block_i, nh, B), lambda i: (i, hb, 0)),
        pl.BlockSpec((nh, B), lambda i: (0, 0)),
    ]
    args = [iembT, uembT]
    body = _bmm_body0
    if acc is not None:
        in_specs.append(pl.BlockSpec((block_i, B), lambda i: (i, 0)))
        args.append(acc)
        body = _bmm_body1
    return pl.pallas_call(
        body,
        grid=(I // block_i,),
        in_specs=in_specs,
        out_specs=pl.BlockSpec((block_i, B), lambda i: (i, 0)),
        out_shape=jax.ShapeDtypeStruct((I, B), jnp.float32),
    )(*args)


def kernel(userid_input, iemb, uembedding_weight):
    idx = userid_input.reshape(-1)
    tableT = uembedding_weight.T
    iembT = jnp.transpose(iemb, (1, 2, 0))
    uembT0 = _make_sc_gather(0, _NH)(idx, tableT)
    uembT1 = _make_sc_gather(_NH, _NH)(idx, tableT)
    predT0 = _tc_bmm(iembT, uembT0, 0)
    predT = _tc_bmm(iembT, uembT1, 1, acc=predT0)
    return predT.T


# speedup vs baseline: 1.0214x; 1.0214x over previous
import functools

import jax
import jax.numpy as jnp
from jax import lax
from jax.experimental import pallas as pl
from jax.experimental.pallas import tpu as pltpu
from jax.experimental.pallas import tpu_sc as plsc

_B = 4096
_I = 200
_H = 64
_LANES = 16
_NSLOT = 8
_NH = 32


def _make_sc_gather(h0, nh):
    info = plsc.get_sparse_core_info()
    NC, NS = info.num_cores, info.num_subcores
    NW = NC * NS
    bpw = _B // NW
    mesh = plsc.VectorSubcoreMesh(core_axis_name="c", subcore_axis_name="s")

    @functools.partial(
        pl.kernel,
        mesh=mesh,
        out_type=jax.ShapeDtypeStruct((nh, _B), jnp.float32),
        scratch_types=[
            pltpu.VMEM((bpw,), jnp.int32),
            pltpu.VMEM((_NSLOT, nh, 128), jnp.float32),
            pltpu.VMEM_SHARED((NS, nh, 128), jnp.float32),
            pltpu.SemaphoreType.DMA((_NSLOT,)),
            pltpu.SemaphoreType.DMA((_NSLOT,)),
        ],
    )
    def gather_kernel(
        idx_hbm, tableT_hbm, out_hbm, idx_v, ring_v, sh_v, tsems, csems
    ):
        cid = lax.axis_index("c")
        sid = lax.axis_index("s")
        wid = sid * NC + cid
        base = wid * bpw
        my_sh = sh_v.at[sid]

        pltpu.sync_copy(idx_hbm.at[pl.ds(base, bpw)], idx_v)

        def fire_tile(u, k):
            start = pl.multiple_of((u >> 7) * 128, 128)
            pltpu.async_copy(
                tableT_hbm.at[pl.ds(h0, nh), pl.ds(start, 128)],
                ring_v.at[k],
                tsems.at[k],
            )

        def wait_tile(k):
            pltpu.make_async_copy(
                tableT_hbm.at[pl.ds(h0, nh), pl.ds(0, 128)],
                ring_v.at[k],
                tsems.at[k],
            ).wait()

        def fire_col(u, b, k):
            pltpu.async_copy(
                ring_v.at[k].at[:, pl.ds(u & 127, 1)],
                my_sh.at[:, pl.ds(b, 1)],
                csems.at[k],
            )

        def wait_col(k):
            pltpu.make_async_copy(
                ring_v.at[k].at[:, pl.ds(0, 1)],
                my_sh.at[:, pl.ds(0, 1)],
                csems.at[k],
            ).wait()

        vs = [idx_v[pl.ds(g * _LANES, _LANES)] for g in range(bpw // _LANES)]

        def u(b):
            return vs[b // _LANES][b % _LANES]

        _LAG = 6
        for b in range(bpw + _LAG):
            if b < bpw:
                if b >= _NSLOT:
                    wait_col(b % _NSLOT)
                fire_tile(u(b), b % _NSLOT)
            if b >= _LAG:
                bb = b - _LAG
                wait_tile(bb % _NSLOT)
                fire_col(u(bb), bb, bb % _NSLOT)
        for k in range(_NSLOT):
            wait_col(k)

        pltpu.sync_copy(my_sh, out_hbm.at[:, pl.ds(base, bpw)])

    return gather_kernel


def _bmm_body0(iembT_ref, uemb_ref, out_ref):
    out_ref[...] = jnp.sum(iembT_ref[...] * uemb_ref[...][None, :, :], axis=1)


def _bmm_body1(iembT_ref, uemb_ref, acc_ref, out_ref):
    out_ref[...] = acc_ref[...] + jnp.sum(
        iembT_ref[...] * uemb_ref[...][None, :, :], axis=1
    )


def _tc_bmm(iembT, uembT, hb, acc=None, block_i=8):
    I, H, B = iembT.shape
    nh = uembT.shape[0]
    in_specs = [
        pl.BlockSpec((block_i, nh, B), lambda i: (i, hb, 0)),
        pl.BlockSpec((nh, B), lambda i: (0, 0)),
    ]
    args = [iembT, uembT]
    body = _bmm_body0
    if acc is not None:
        in_specs.append(pl.BlockSpec((block_i, B), lambda i: (i, 0)))
        args.append(acc)
        body = _bmm_body1
    return pl.pallas_call(
        body,
        grid=(I // block_i,),
        in_specs=in_specs,
        out_specs=pl.BlockSpec((block_i, B), lambda i: (i, 0)),
        out_shape=jax.ShapeDtypeStruct((I, B), jnp.float32),
    )(*args)


def kernel(userid_input, iemb, uembedding_weight):
    idx = userid_input.reshape(-1)
    tableT = uembedding_weight.T
    iembT = jnp.transpose(iemb, (1, 2, 0))
    uembT = _make_sc_gather(0, _H)(idx, tableT)
    predT = _tc_bmm(iembT, uembT, 0, block_i=8)
    return predT.T


# speedup vs baseline: 1.0261x; 1.0046x over previous
import functools

import jax
import jax.numpy as jnp
from jax import lax
from jax.experimental import pallas as pl
from jax.experimental.pallas import tpu as pltpu
from jax.experimental.pallas import tpu_sc as plsc

_B = 4096
_I = 200
_H = 64
_LANES = 16
_NSLOT = 8
_NH = 32


def _make_sc_gather(h0, nh):
    info = plsc.get_sparse_core_info()
    NC, NS = info.num_cores, info.num_subcores
    NW = NC * NS
    bpw = _B // NW
    mesh = plsc.VectorSubcoreMesh(core_axis_name="c", subcore_axis_name="s")

    @functools.partial(
        pl.kernel,
        mesh=mesh,
        out_type=jax.ShapeDtypeStruct((nh, _B), jnp.float32),
        scratch_types=[
            pltpu.VMEM((bpw,), jnp.int32),
            pltpu.VMEM((_NSLOT, nh, 128), jnp.float32),
            pltpu.VMEM_SHARED((NS, nh, 128), jnp.float32),
            pltpu.SemaphoreType.DMA((_NSLOT,)),
            pltpu.SemaphoreType.DMA((_NSLOT,)),
        ],
    )
    def gather_kernel(
        idx_hbm, tableT_hbm, out_hbm, idx_v, ring_v, sh_v, tsems, csems
    ):
        cid = lax.axis_index("c")
        sid = lax.axis_index("s")
        wid = sid * NC + cid
        base = wid * bpw
        my_sh = sh_v.at[sid]

        pltpu.sync_copy(idx_hbm.at[pl.ds(base, bpw)], idx_v)

        def fire_tile(u, k):
            start = pl.multiple_of((u >> 7) * 128, 128)
            pltpu.async_copy(
                tableT_hbm.at[pl.ds(h0, nh), pl.ds(start, 128)],
                ring_v.at[k],
                tsems.at[k],
            )

        def wait_tile(k):
            pltpu.make_async_copy(
                tableT_hbm.at[pl.ds(h0, nh), pl.ds(0, 128)],
                ring_v.at[k],
                tsems.at[k],
            ).wait()

        def fire_col(u, b, k):
            pltpu.async_copy(
                ring_v.at[k].at[:, pl.ds(u & 127, 1)],
                my_sh.at[:, pl.ds(b, 1)],
                csems.at[k],
            )

        def wait_col(k):
            pltpu.make_async_copy(
                ring_v.at[k].at[:, pl.ds(0, 1)],
                my_sh.at[:, pl.ds(0, 1)],
                csems.at[k],
            ).wait()

        vs = [idx_v[pl.ds(g * _LANES, _LANES)] for g in range(bpw // _LANES)]

        def u(b):
            return vs[b // _LANES][b % _LANES]

        _LAG = 6
        for b in range(bpw + _LAG):
            if b < bpw:
                if b >= _NSLOT:
                    wait_col(b % _NSLOT)
                fire_tile(u(b), b % _NSLOT)
            if b >= _LAG:
                bb = b - _LAG
                wait_tile(bb % _NSLOT)
                fire_col(u(bb), bb, bb % _NSLOT)
        for k in range(_NSLOT):
            wait_col(k)

        pltpu.sync_copy(my_sh, out_hbm.at[:, pl.ds(base, bpw)])

    return gather_kernel


def _bmm_body0(iembT_ref, uemb_ref, out_ref):
    out_ref[...] = jnp.sum(iembT_ref[...] * uemb_ref[...][None, :, :], axis=1)


def _bmm_body1(iembT_ref, uemb_ref, acc_ref, out_ref):
    out_ref[...] = acc_ref[...] + jnp.sum(
        iembT_ref[...] * uemb_ref[...][None, :, :], axis=1
    )


def _tc_bmm(iembT, uembT, hb, acc=None, block_i=8):
    I, H, B = iembT.shape
    nh = uembT.shape[0]
    in_specs = [
        pl.BlockSpec((block_i, nh, B), lambda i: (i, hb, 0)),
        pl.BlockSpec((nh, B), lambda i: (0, 0)),
    ]
    args = [iembT, uembT]
    body = _bmm_body0
    if acc is not None:
        in_specs.append(pl.BlockSpec((block_i, B), lambda i: (i, 0)))
        args.append(acc)
        body = _bmm_body1
    return pl.pallas_call(
        body,
        grid=(I // block_i,),
        in_specs=in_specs,
        out_specs=pl.BlockSpec((block_i, B), lambda i: (i, 0)),
        out_shape=jax.ShapeDtypeStruct((I, B), jnp.float32),
    )(*args)


def kernel(userid_input, iemb, uembedding_weight):
    idx = userid_input.reshape(-1)
    tableT = uembedding_weight.T
    iembT = jnp.transpose(iemb, (1, 2, 0))
    uembT = _make_sc_gather(0, _H)(idx, tableT)
    predT = _tc_bmm(iembT, uembT, 0, block_i=16)
    return predT.T
